# Initial kernel scaffold; baseline (speedup 1.0000x reference)
import jax
import jax.numpy as jnp
from jax.experimental import pallas as pl


def kernel(xyz, feat, W, b, gamma, beta):
    raise NotImplementedError("write your pallas kernel here")


# speedup vs baseline: 6.5874x; 6.5874x over previous
import functools

import jax
import jax.numpy as jnp
from jax import lax
from jax.experimental import pallas as pl
from jax.experimental.pallas import tpu as pltpu

KNN_K = 32


def _knn_body(xr_ref, yr_ref, xc_ref, yc_ref, idx_ref, *, n, k):
    xr = xr_ref[0]
    yr = yr_ref[0]
    xc = xc_ref[0]
    yc = yc_ref[0]
    bxr = xr.astype(jnp.bfloat16).astype(jnp.float32)
    byr = yr.astype(jnp.bfloat16).astype(jnp.float32)
    bxc = xc.astype(jnp.bfloat16).astype(jnp.float32)
    byc = yc.astype(jnp.bfloat16).astype(jnp.float32)
    inner2 = 2.0 * (bxc * bxr + byc * byr)
    xxr = xr * xr + yr * yr
    xxc = xc * xc + yc * yc
    neg = inner2 - xxc - xxr
    iota = lax.broadcasted_iota(jnp.int32, (n, n), 1)
    cols = []
    for _ in range(k):
        m = jnp.max(neg, axis=1, keepdims=True)
        am = jnp.min(jnp.where(neg == m, iota, n), axis=1,
                     keepdims=True)
        cols.append(am)
        neg = jnp.where(iota == am, -jnp.inf, neg)
    idx_ref[0] = jnp.concatenate(cols, axis=1)


def _g_body(mit_ref, featT_ref, g_ref, *, n, k, rb):
    mit = mit_ref[0]
    iota = lax.broadcasted_iota(jnp.int32, (rb, n), 1)
    cnt = jnp.zeros((rb, n), jnp.int32)
    for j in range(k):
        cnt = cnt + jnp.where(mit[:, j:j + 1] == iota, 1, 0)
    ct = cnt.astype(jnp.float32)
    g_ref[0] = jnp.dot(ct, featT_ref[0], preferred_element_type=jnp.float32)


def _trans_body(g_ref, feat_ref, wt_ref, b_ref, trans_ref, s1_ref, s2_ref,
                *, k):
    lap = feat_ref[0] - g_ref[0] * (1.0 / k)
    trans = jnp.dot(lap, wt_ref[...], preferred_element_type=jnp.float32)
    trans = trans + b_ref[...]
    trans_ref[0] = trans
    s1_ref[0] = jnp.sum(trans, axis=1, keepdims=True)
    s2_ref[0] = jnp.sum(trans * trans, axis=1, keepdims=True)


def _final_body(trans_ref, feat_ref, s1_ref, s2_ref, g_ref, be_ref, out_ref,
                *, nb, f):
    b = s1_ref.shape[0]
    s1 = s1_ref[0]
    s2 = s2_ref[0]
    for bi in range(1, b):
        s1 = s1 + s1_ref[bi]
        s2 = s2 + s2_ref[bi]
    denom = 1.0 / (b * f)
    mean = s1 * denom
    var = s2 * denom - mean * mean
    rstd = lax.rsqrt(var + 1e-5)
    trans = trans_ref[0]
    t = (trans - mean) * rstd * g_ref[0] + be_ref[0]
    out_ref[0] = feat_ref[0] + jnp.maximum(t, 0.0)


def kernel(xyz, feat, W, b, gamma, beta):
    B, N, _ = xyz.shape
    F = feat.shape[2]
    K = KNN_K
    RB = min(256, N)
    NB = N // RB

    xr = xyz[:, :, 0].reshape(B, 1, N)
    yr = xyz[:, :, 1].reshape(B, 1, N)
    xc = xyz[:, :, 0].reshape(B, N, 1)
    yc = xyz[:, :, 1].reshape(B, N, 1)

    idx = pl.pallas_call(
        functools.partial(_knn_body, n=N, k=K),
        grid=(B,),
        in_specs=[
            pl.BlockSpec((1, 1, N), lambda bi: (bi, 0, 0)),
            pl.BlockSpec((1, 1, N), lambda bi: (bi, 0, 0)),
            pl.BlockSpec((1, N, 1), lambda bi: (bi, 0, 0)),
            pl.BlockSpec((1, N, 1), lambda bi: (bi, 0, 0)),
        ],
        out_specs=pl.BlockSpec((1, N, K), lambda bi: (bi, 0, 0)),
        out_shape=jax.ShapeDtypeStruct((B, N, K), jnp.int32),
    )(xr, yr, xc, yc)

    mit = idx.reshape(B, K, N).transpose(0, 2, 1)
    featT = feat.transpose(0, 2, 1)
    wt = W.T
    b2 = b.reshape(1, F)

    g = pl.pallas_call(
        functools.partial(_g_body, n=N, k=K, rb=RB),
        grid=(B, NB),
        in_specs=[
            pl.BlockSpec((1, RB, K), lambda bi, i: (bi, i, 0)),
            pl.BlockSpec((1, N, F), lambda bi, i: (bi, 0, 0)),
        ],
        out_specs=pl.BlockSpec((1, RB, F), lambda bi, i: (bi, i, 0)),
        out_shape=jax.ShapeDtypeStruct((B, N, F), jnp.float32),
    )(mit, featT)

    trans, s1, s2 = pl.pallas_call(
        functools.partial(_trans_body, k=K),
        grid=(B, NB),
        in_specs=[
            pl.BlockSpec((1, RB, F), lambda bi, i: (bi, i, 0)),
            pl.BlockSpec((1, RB, F), lambda bi, i: (bi, i, 0)),
            pl.BlockSpec((F, F), lambda bi, i: (0, 0)),
            pl.BlockSpec((1, F), lambda bi, i: (0, 0)),
        ],
        out_specs=[
            pl.BlockSpec((1, RB, F), lambda bi, i: (bi, i, 0)),
            pl.BlockSpec((1, RB, 1), lambda bi, i: (bi, i, 0)),
            pl.BlockSpec((1, RB, 1), lambda bi, i: (bi, i, 0)),
        ],
        out_shape=[
            jax.ShapeDtypeStruct((B, N, F), jnp.float32),
            jax.ShapeDtypeStruct((B, N, 1), jnp.float32),
            jax.ShapeDtypeStruct((B, N, 1), jnp.float32),
        ],
    )(g, feat, wt, b2)

    g3 = gamma.reshape(1, N, 1)
    be3 = beta.reshape(1, N, 1)

    out = pl.pallas_call(
        functools.partial(_final_body, nb=NB, f=F),
        grid=(B, NB),
        in_specs=[
            pl.BlockSpec((1, RB, F), lambda bi, i: (bi, i, 0)),
            pl.BlockSpec((1, RB, F), lambda bi, i: (bi, i, 0)),
            pl.BlockSpec((B, RB, 1), lambda bi, i: (0, i, 0)),
            pl.BlockSpec((B, RB, 1), lambda bi, i: (0, i, 0)),
            pl.BlockSpec((1, RB, 1), lambda bi, i: (0, i, 0)),
            pl.BlockSpec((1, RB, 1), lambda bi, i: (0, i, 0)),
        ],
        out_specs=pl.BlockSpec((1, RB, F), lambda bi, i: (bi, i, 0)),
        out_shape=jax.ShapeDtypeStruct((B, N, F), jnp.float32),
    )(trans, feat, s1, s2, g3, be3)
    return out
